# baseline (speedup 1.0000x reference)
import functools

import jax
import jax.numpy as jnp
from jax import lax
from jax.experimental import pallas as pl
from jax.experimental.pallas import tpu as pltpu
from jax.experimental.pallas import tpu_sc as plsc

_P = 0.5
_MISSING = 0.0
_N = 16777216

_COIN = 0.5302608013153076
_LO = 1820685
_HI = 10756808
_APPLY = _COIN < _P

_NC = 2
_NS = 16
_NW = _NC * _NS
_S = _N // _NW
_CH = 16384
_LANES = 16


_CCH = 32768
_NBUF = 3


def _build_copy():
    mesh = plsc.VectorSubcoreMesh(core_axis_name="c", subcore_axis_name="s")
    nch = _S // _CCH

    @functools.partial(
        pl.kernel,
        mesh=mesh,
        out_type=jax.ShapeDtypeStruct((_N,), jnp.float32),
        scratch_types=[pltpu.VMEM((_CCH,), jnp.float32)] * _NBUF
                      + [pltpu.SemaphoreType.DMA] * (2 * _NBUF),
    )
    def k(arr_hbm, out_hbm, *scratch):
        bufs = scratch[:_NBUF]
        sin, sout = scratch[_NBUF:2 * _NBUF], scratch[2 * _NBUF:]
        wid = lax.axis_index("s") * _NC + lax.axis_index("c")
        base = wid * _S

        def load(j, b):
            return pltpu.make_async_copy(
                arr_hbm.at[pl.ds(base + j * _CCH, _CCH)], bufs[b], sin[b])

        def store(j, b):
            return pltpu.make_async_copy(
                bufs[b], out_hbm.at[pl.ds(base + j * _CCH, _CCH)], sout[b])

        for j in range(_NBUF):
            load(j, j).start()
        for j in range(nch):
            b = j % _NBUF
            load(j, b).wait()
            store(j, b).start()
            pj = j + _NBUF - 1
            if _NBUF <= pj < nch:
                pb = pj % _NBUF
                store(pj - _NBUF, pb).wait()
                load(pj, pb).start()
        for j in range(max(0, nch - _NBUF), nch):
            store(j, j % _NBUF).wait()

    return k


def _build_masked():
    mesh = plsc.VectorSubcoreMesh(core_axis_name="c", subcore_axis_name="s")

    @functools.partial(
        pl.kernel,
        mesh=mesh,
        out_type=jax.ShapeDtypeStruct((_N,), jnp.float32),
        scratch_types=[pltpu.VMEM((_CH,), jnp.float32)],
    )
    def k(arr_hbm, out_hbm, buf):
        wid = lax.axis_index("s") * _NC + lax.axis_index("c")
        base = wid * _S

        def chunk_body(j, _):
            off = base + j * _CH
            pltpu.sync_copy(arr_hbm.at[pl.ds(off, _CH)], buf)

            def vec_body(i, _):
                idx = off + i * _LANES + lax.iota(jnp.int32, _LANES)
                x = buf[pl.ds(i * _LANES, _LANES)]
                m = (idx >= _LO) & (idx < _HI)
                buf[pl.ds(i * _LANES, _LANES)] = jnp.where(
                    m, jnp.float32(_MISSING), x)
                return 0

            lax.fori_loop(0, _CH // _LANES, vec_body, 0)
            pltpu.sync_copy(buf, out_hbm.at[pl.ds(off, _CH)])
            return 0

        lax.fori_loop(0, _S // _CH, chunk_body, 0)

    return k


def _build_copy_tc():
    rows = 256

    def body(x_ref, o_ref):
        o_ref[...] = x_ref[...]

    call = pl.pallas_call(
        body,
        grid=(4096 // rows,),
        in_specs=[pl.BlockSpec((rows, 4096), lambda i: (i, 0))],
        out_specs=pl.BlockSpec((rows, 4096), lambda i: (i, 0)),
        out_shape=jax.ShapeDtypeStruct((4096, 4096), jnp.float32),
    )

    def k(arr):
        return call(arr.reshape(4096, 4096)).reshape(-1)

    return k


_IMPL = _build_masked() if _APPLY else _build_copy_tc()


def kernel(arr):
    out = _IMPL(arr)
    key = jax.random.key(42)
    k_coin, k_bounds = jax.random.split(key)
    coin = jax.random.uniform(k_coin, (), dtype=jnp.float32)
    bounds = jax.random.randint(k_bounds, (2,), 0, _N)
    lo = jnp.min(bounds)
    hi = jnp.max(bounds)
    idx = jnp.arange(_N)
    in_span = (idx >= lo) & (idx < hi)
    out = jnp.where((coin < _P) & in_span,
                    jnp.asarray(_MISSING, dtype=out.dtype), out)
    return out


# speedup vs baseline: 1.9861x; 1.9861x over previous
import functools

import jax
import jax.numpy as jnp
from jax import lax
from jax.experimental import pallas as pl
from jax.experimental.pallas import tpu as pltpu
from jax.experimental.pallas import tpu_sc as plsc

_P = 0.5
_MISSING = 0.0
_N = 16777216

_COIN = 0.5302608013153076
_LO = 1820685
_HI = 10756808
_APPLY = _COIN < _P

_NC = 2
_NS = 16
_NW = _NC * _NS
_S = _N // _NW
_CH = 16384
_LANES = 16

_CCH = 16384
_NBUF = 6


def _build_copy():
    mesh = plsc.VectorSubcoreMesh(core_axis_name="c", subcore_axis_name="s")
    nch = _S // _CCH

    @functools.partial(
        pl.kernel,
        mesh=mesh,
        out_type=jax.ShapeDtypeStruct((_N,), jnp.float32),
        scratch_types=[pltpu.VMEM((_CCH,), jnp.float32)] * _NBUF
                      + [pltpu.SemaphoreType.DMA] * (2 * _NBUF),
    )
    def k(arr_hbm, out_hbm, *scratch):
        bufs = scratch[:_NBUF]
        sin, sout = scratch[_NBUF:2 * _NBUF], scratch[2 * _NBUF:]
        wid = lax.axis_index("s") * _NC + lax.axis_index("c")

        def load(j, b):
            off = (j * _NW + wid) * _CCH
            return pltpu.make_async_copy(
                arr_hbm.at[pl.ds(off, _CCH)], bufs[b], sin[b])

        def store(j, b):
            off = (j * _NW + wid) * _CCH
            return pltpu.make_async_copy(
                bufs[b], out_hbm.at[pl.ds(off, _CCH)], sout[b])

        for j in range(_NBUF):
            load(j, j).start()
        for j in range(nch):
            b = j % _NBUF
            load(j, b).wait()
            store(j, b).start()
            pj = j + _NBUF - 1
            if _NBUF <= pj < nch:
                pb = pj % _NBUF
                store(pj - _NBUF, pb).wait()
                load(pj, pb).start()
        for j in range(max(0, nch - _NBUF), nch):
            store(j, j % _NBUF).wait()

    return k


def _build_masked():
    mesh = plsc.VectorSubcoreMesh(core_axis_name="c", subcore_axis_name="s")

    @functools.partial(
        pl.kernel,
        mesh=mesh,
        out_type=jax.ShapeDtypeStruct((_N,), jnp.float32),
        scratch_types=[pltpu.VMEM((_CH,), jnp.float32)],
    )
    def k(arr_hbm, out_hbm, buf):
        wid = lax.axis_index("s") * _NC + lax.axis_index("c")
        base = wid * _S

        def chunk_body(j, _):
            off = base + j * _CH
            pltpu.sync_copy(arr_hbm.at[pl.ds(off, _CH)], buf)

            def vec_body(i, _):
                idx = off + i * _LANES + lax.iota(jnp.int32, _LANES)
                x = buf[pl.ds(i * _LANES, _LANES)]
                m = (idx >= _LO) & (idx < _HI)
                buf[pl.ds(i * _LANES, _LANES)] = jnp.where(
                    m, jnp.float32(_MISSING), x)
                return 0

            lax.fori_loop(0, _CH // _LANES, vec_body, 0)
            pltpu.sync_copy(buf, out_hbm.at[pl.ds(off, _CH)])
            return 0

        lax.fori_loop(0, _S // _CH, chunk_body, 0)

    return k


_IMPL = _build_masked() if _APPLY else _build_copy()


def kernel(arr):
    out = _IMPL(arr)
    key = jax.random.key(42)
    k_coin, k_bounds = jax.random.split(key)
    coin = jax.random.uniform(k_coin, (), dtype=jnp.float32)
    bounds = jax.random.randint(k_bounds, (2,), 0, _N)
    lo = jnp.min(bounds)
    hi = jnp.max(bounds)
    idx = jnp.arange(_N)
    in_span = (idx >= lo) & (idx < hi)
    out = jnp.where((coin < _P) & in_span,
                    jnp.asarray(_MISSING, dtype=out.dtype), out)
    return out
